# Initial kernel scaffold; baseline (speedup 1.0000x reference)
import jax
import jax.numpy as jnp
from jax.experimental import pallas as pl


def kernel(single, pair, coords, contact_ss, ln_w, ln_b, Wq, Wk, Wv, W_eb, Wo, bo):
    raise NotImplementedError("write your pallas kernel here")


# speedup vs baseline: 169.4073x; 169.4073x over previous
import jax
import jax.numpy as jnp
from jax.experimental import pallas as pl

_K_KNN = 12
_K_SS = 4


def _body(single_ref, pair0_ref, coords_ref, coordsT_ref, css_ref,
          ln_w_ref, ln_b_ref, wqT_ref, wkT_ref, wvT_ref, webT_ref,
          woT_ref, bo_ref, out_ref):
    f32 = jnp.float32
    L, D = single_ref.shape[1], single_ref.shape[2]
    H = webT_ref.shape[1]
    Dh = D // H
    E = _K_KNN + _K_SS
    scale = f32(Dh ** -0.5)

    x = single_ref[0]
    mu = jnp.mean(x, axis=1, keepdims=True)
    xc = x - mu
    var = jnp.mean(xc * xc, axis=1, keepdims=True)
    xn = xc / jnp.sqrt(var + 1e-5) * ln_w_ref[...] + ln_b_ref[...]

    q = jnp.dot(xn, wqT_ref[...], preferred_element_type=f32)
    k = jnp.dot(xn, wkT_ref[...], preferred_element_type=f32)
    v = jnp.dot(xn, wvT_ref[...], preferred_element_type=f32)

    lanes = jax.lax.broadcasted_iota(jnp.int32, (L, L), 1)
    rows = jax.lax.broadcasted_iota(jnp.int32, (L, L), 0)

    s2 = None
    for a in range(3):
        dc = coords_ref[0, :, a:a + 1] - coordsT_ref[0, a:a + 1, :]
        s2 = dc * dc if s2 is None else s2 + dc * dc
    d = jnp.sqrt(s2) + jnp.where(rows == lanes, f32(1e9), f32(0.0))

    neg_inf = f32(-jnp.inf)

    def topk_cols(vals, kk):
        cols = []
        vc = vals
        for _ in range(kk):
            m = jnp.max(vc, axis=1, keepdims=True)
            am = jnp.min(jnp.where(vc == m, lanes, jnp.int32(L)),
                         axis=1, keepdims=True)
            cols.append(am)
            vc = jnp.where(lanes == am, neg_inf, vc)
        return cols

    cols = topk_cols(-d, _K_KNN) + topk_cols(css_ref[0], _K_SS)

    catcol = jnp.concatenate(cols, axis=0)
    lanesE = jax.lax.broadcasted_iota(jnp.int32, (E * L, L), 1)
    rowsE = jax.lax.broadcasted_iota(jnp.int32, (E * L, L), 0)
    mstack = (catcol == lanesE).astype(f32)

    p0 = pair0_ref[0, 0]
    r_all = jnp.dot(mstack, p0, preferred_element_type=f32)
    selcol = 16 * ((rowsE % L) % 4) + rowsE // L
    t_sel = r_all * (lanesE == selcol).astype(f32)
    bmat = jnp.dot(t_sel, webT_ref[...], preferred_element_type=f32)
    r2 = jax.lax.broadcasted_iota(jnp.int32, (E, E * L), 1)
    e2i = jax.lax.broadcasted_iota(jnp.int32, (E, E * L), 0)
    pool = (((r2 % L) // 4) == e2i).astype(f32)
    bias_t = jnp.dot(pool, bmat, preferred_element_type=f32)
    bmax = jnp.max(bias_t, axis=0, keepdims=True)
    expb = jnp.exp(bias_t - bmax)

    outs = []
    for h in range(H):
        g = None
        for e in range(E):
            me = mstack[e * L:(e + 1) * L, :]
            w = expb[e:e + 1, h:h + 1]
            g = me * w if g is None else g + me * w
        sh = jax.lax.dot_general(
            q[:, h * Dh:(h + 1) * Dh], k[:, h * Dh:(h + 1) * Dh],
            (((1,), (1,)), ((), ())), preferred_element_type=f32)
        t = sh * scale + jnp.log(g)
        m = jnp.max(t, axis=1, keepdims=True)
        p = jnp.exp(t - m)
        attn = p / jnp.sum(p, axis=1, keepdims=True)
        outs.append(jnp.dot(attn, v[:, h * Dh:(h + 1) * Dh],
                            preferred_element_type=f32))
    ctx = jnp.concatenate(outs, axis=1)
    out_ref[0] = x + jnp.dot(ctx, woT_ref[...],
                             preferred_element_type=f32) + bo_ref[...]


def kernel(single, pair, coords, contact_ss, ln_w, ln_b, Wq, Wk, Wv, W_eb,
           Wo, bo):
    B, L, D = single.shape
    DP = pair.shape[-1]
    coordsT = jnp.transpose(coords, (0, 2, 1))
    args = (single, pair, coords, coordsT, contact_ss,
            ln_w.reshape(1, D), ln_b.reshape(1, D),
            Wq.T, Wk.T, Wv.T, W_eb.T, Wo.T, bo.reshape(1, D))
    in_specs = [
        pl.BlockSpec((1, L, D), lambda b: (b, 0, 0)),
        pl.BlockSpec((1, 1, L, DP), lambda b: (b, 0, 0, 0)),
        pl.BlockSpec((1, L, 3), lambda b: (b, 0, 0)),
        pl.BlockSpec((1, 3, L), lambda b: (b, 0, 0)),
        pl.BlockSpec((1, L, L), lambda b: (b, 0, 0)),
        pl.BlockSpec((1, D), lambda b: (0, 0)),
        pl.BlockSpec((1, D), lambda b: (0, 0)),
        pl.BlockSpec((D, D), lambda b: (0, 0)),
        pl.BlockSpec((D, D), lambda b: (0, 0)),
        pl.BlockSpec((D, D), lambda b: (0, 0)),
        pl.BlockSpec((DP, W_eb.shape[0]), lambda b: (0, 0)),
        pl.BlockSpec((D, D), lambda b: (0, 0)),
        pl.BlockSpec((1, D), lambda b: (0, 0)),
    ]
    return pl.pallas_call(
        _body,
        grid=(B,),
        in_specs=in_specs,
        out_specs=pl.BlockSpec((1, L, D), lambda b: (b, 0, 0)),
        out_shape=jax.ShapeDtypeStruct((B, L, D), jnp.float32),
    )(*args)


# speedup vs baseline: 289.6764x; 1.7099x over previous
import jax
import jax.numpy as jnp
from jax.experimental import pallas as pl

_K_KNN = 12
_K_SS = 4
_BB = 8


def _body(single_ref, pair0_ref, coords_ref, coordsT_ref, css_ref,
          ln_w_ref, ln_b_ref, wqT_ref, wkT_ref, wvT_ref, webT_ref,
          woT_ref, bo_ref, out_ref):
    f32 = jnp.float32
    BB, L, D = single_ref.shape
    H = webT_ref.shape[1]
    Dh = D // H
    E = _K_KNN + _K_SS
    R = BB * L
    scale = f32(Dh ** -0.5)

    x2 = single_ref[...].reshape(R, D)
    mu = jnp.mean(x2, axis=1, keepdims=True)
    xc = x2 - mu
    var = jnp.mean(xc * xc, axis=1, keepdims=True)
    xn = xc / jnp.sqrt(var + 1e-5) * ln_w_ref[...] + ln_b_ref[...]

    q = jnp.dot(xn, wqT_ref[...], preferred_element_type=f32)
    k = jnp.dot(xn, wkT_ref[...], preferred_element_type=f32)
    v = jnp.dot(xn, wvT_ref[...], preferred_element_type=f32)

    lanesR = jax.lax.broadcasted_iota(jnp.int32, (R, L), 1)
    lanesL = jax.lax.broadcasted_iota(jnp.int32, (L, L), 1)
    rowsL = jax.lax.broadcasted_iota(jnp.int32, (L, L), 0)
    eyeL = jnp.where(rowsL == lanesL, f32(1e9), f32(0.0))

    negd_parts = []
    for bb in range(BB):
        s2 = None
        for a in range(3):
            dc = coords_ref[bb, :, a:a + 1] - coordsT_ref[bb, a:a + 1, :]
            s2 = dc * dc if s2 is None else s2 + dc * dc
        negd_parts.append(-(jnp.sqrt(s2) + eyeL))
    negd = jnp.concatenate(negd_parts, axis=0)
    css2 = css_ref[...].reshape(R, L)

    neg_inf = f32(-jnp.inf)

    def topk_cols(vals, kk):
        cols = []
        vc = vals
        for _ in range(kk):
            m = jnp.max(vc, axis=1, keepdims=True)
            am = jnp.min(jnp.where(vc == m, lanesR, jnp.int32(L)),
                         axis=1, keepdims=True)
            cols.append(am)
            vc = jnp.where(lanesR == am, neg_inf, vc)
        return cols

    cols = topk_cols(negd, _K_KNN) + topk_cols(css2, _K_SS)
    masks = [(c == lanesR).astype(f32) for c in cols]

    lanesEL = jax.lax.broadcasted_iota(jnp.int32, (E * L, L), 1)
    rowsEL = jax.lax.broadcasted_iota(jnp.int32, (E * L, L), 0)
    selmask = (lanesEL == (16 * ((rowsEL % L) % 4) + rowsEL // L)).astype(f32)
    r2 = jax.lax.broadcasted_iota(jnp.int32, (E, E * L), 1)
    e2i = jax.lax.broadcasted_iota(jnp.int32, (E, E * L), 0)
    pool = (((r2 % L) // 4) == e2i).astype(f32)

    expb = []
    for bb in range(BB):
        mstack = jnp.concatenate(
            [masks[e][bb * L:(bb + 1) * L, :] for e in range(E)], axis=0)
        p0 = pair0_ref[bb, 0]
        r_all = jnp.dot(mstack, p0, preferred_element_type=f32)
        bmat = jnp.dot(r_all * selmask, webT_ref[...],
                       preferred_element_type=f32)
        bias_t = jnp.dot(pool, bmat, preferred_element_type=f32)
        bmax = jnp.max(bias_t, axis=0, keepdims=True)
        expb.append(jnp.exp(bias_t - bmax))

    ctx_parts = []
    for bb in range(BB):
        sl = slice(bb * L, (bb + 1) * L)
        outs = []
        for h in range(H):
            g = None
            for e in range(E):
                w = expb[bb][e:e + 1, h:h + 1]
                me = masks[e][sl, :]
                g = me * w if g is None else g + me * w
            sh = jax.lax.dot_general(
                q[sl, h * Dh:(h + 1) * Dh], k[sl, h * Dh:(h + 1) * Dh],
                (((1,), (1,)), ((), ())), preferred_element_type=f32)
            t = sh * scale + jnp.log(g)
            m = jnp.max(t, axis=1, keepdims=True)
            p = jnp.exp(t - m)
            attn = p / jnp.sum(p, axis=1, keepdims=True)
            outs.append(jnp.dot(attn, v[sl, h * Dh:(h + 1) * Dh],
                                preferred_element_type=f32))
        ctx_parts.append(jnp.concatenate(outs, axis=1))
    ctx = jnp.concatenate(ctx_parts, axis=0)
    out2 = x2 + jnp.dot(ctx, woT_ref[...],
                        preferred_element_type=f32) + bo_ref[...]
    out_ref[...] = out2.reshape(BB, L, D)


def kernel(single, pair, coords, contact_ss, ln_w, ln_b, Wq, Wk, Wv, W_eb,
           Wo, bo):
    B, L, D = single.shape
    DP = pair.shape[-1]
    BB = _BB
    coordsT = jnp.transpose(coords, (0, 2, 1))
    args = (single, pair, coords, coordsT, contact_ss,
            ln_w.reshape(1, D), ln_b.reshape(1, D),
            Wq.T, Wk.T, Wv.T, W_eb.T, Wo.T, bo.reshape(1, D))
    in_specs = [
        pl.BlockSpec((BB, L, D), lambda b: (b, 0, 0)),
        pl.BlockSpec((BB, 1, L, DP), lambda b: (b, 0, 0, 0)),
        pl.BlockSpec((BB, L, 3), lambda b: (b, 0, 0)),
        pl.BlockSpec((BB, 3, L), lambda b: (b, 0, 0)),
        pl.BlockSpec((BB, L, L), lambda b: (b, 0, 0)),
        pl.BlockSpec((1, D), lambda b: (0, 0)),
        pl.BlockSpec((1, D), lambda b: (0, 0)),
        pl.BlockSpec((D, D), lambda b: (0, 0)),
        pl.BlockSpec((D, D), lambda b: (0, 0)),
        pl.BlockSpec((D, D), lambda b: (0, 0)),
        pl.BlockSpec((DP, W_eb.shape[0]), lambda b: (0, 0)),
        pl.BlockSpec((D, D), lambda b: (0, 0)),
        pl.BlockSpec((1, D), lambda b: (0, 0)),
    ]
    return pl.pallas_call(
        _body,
        grid=(B // BB,),
        in_specs=in_specs,
        out_specs=pl.BlockSpec((BB, L, D), lambda b: (b, 0, 0)),
        out_shape=jax.ShapeDtypeStruct((B, L, D), jnp.float32),
    )(*args)


# speedup vs baseline: 454.8260x; 1.5701x over previous
import jax
import jax.numpy as jnp
from jax.experimental import pallas as pl

_K_KNN = 12
_K_SS = 4
_BB = 8


def _body(single_ref, pair0_ref, coords_ref, coordsT_ref, css_ref,
          ln_w_ref, ln_b_ref, wqT_ref, wkT_ref, wvT_ref, webT_ref,
          woT_ref, bo_ref, out_ref):
    f32 = jnp.float32
    BB, L, D = single_ref.shape
    H = webT_ref.shape[1]
    Dh = D // H
    E = _K_KNN + _K_SS
    R = BB * L
    scale = f32(Dh ** -0.5)

    x2 = single_ref[...].reshape(R, D)
    mu = jnp.mean(x2, axis=1, keepdims=True)
    xc = x2 - mu
    var = jnp.mean(xc * xc, axis=1, keepdims=True)
    xn = xc / jnp.sqrt(var + 1e-5) * ln_w_ref[...] + ln_b_ref[...]

    q = jnp.dot(xn, wqT_ref[...], preferred_element_type=f32)
    k = jnp.dot(xn, wkT_ref[...], preferred_element_type=f32)
    v = jnp.dot(xn, wvT_ref[...], preferred_element_type=f32)

    lanesR = jax.lax.broadcasted_iota(jnp.int32, (R, L), 1)
    rowsR = jax.lax.broadcasted_iota(jnp.int32, (R, L), 0)
    eyeR = jnp.where((rowsR % L) == lanesR, f32(1e9), f32(0.0))

    s2 = None
    for a in range(3):
        colv = coords_ref[...].reshape(R, 3)[:, a:a + 1]
        rowv = jnp.broadcast_to(coordsT_ref[:, a:a + 1, :],
                                (BB, L, L)).reshape(R, L)
        dc = colv - rowv
        s2 = dc * dc if s2 is None else s2 + dc * dc
    negd = -(jnp.sqrt(s2) + eyeR)
    css2 = css_ref[...].reshape(R, L)

    neg_inf = f32(-jnp.inf)

    def topk_cols(vals, kk):
        cols = []
        vc = vals
        for _ in range(kk):
            am = jnp.argmax(vc, axis=1, keepdims=True).astype(jnp.int32)
            cols.append(am)
            vc = jnp.where(lanesR == am, neg_inf, vc)
        return cols

    both = jnp.concatenate([negd, css2], axis=0)
    lanes2R = jax.lax.broadcasted_iota(jnp.int32, (2 * R, L), 1)
    vc2 = both
    cols2 = []
    for _ in range(_K_SS):
        am = jnp.argmax(vc2, axis=1, keepdims=True).astype(jnp.int32)
        cols2.append(am)
        vc2 = jnp.where(lanes2R == am, neg_inf, vc2)
    knn_cols = [c[:R] for c in cols2] + topk_cols(vc2[:R], _K_KNN - _K_SS)
    ss_cols = [c[R:] for c in cols2]
    cols = knn_cols + ss_cols

    lanesEL = jax.lax.broadcasted_iota(jnp.int32, (E * L, L), 1)
    rowsEL = jax.lax.broadcasted_iota(jnp.int32, (E * L, L), 0)
    selmask = (lanesEL == (16 * ((rowsEL % L) % 4) + rowsEL // L)).astype(f32)
    r2 = jax.lax.broadcasted_iota(jnp.int32, (E, E * L), 1)
    e2i = jax.lax.broadcasted_iota(jnp.int32, (E, E * L), 0)
    pool = (((r2 % L) // 4) == e2i).astype(f32)

    catcols = [jnp.concatenate([cols[e][bb * L:(bb + 1) * L] for e in range(E)],
                               axis=0) for bb in range(BB)]
    mstacks = [(cc == lanesEL).astype(f32) for cc in catcols]
    r_alls = [jnp.dot(mstacks[bb], pair0_ref[bb, 0],
                      preferred_element_type=f32) for bb in range(BB)]
    bmats = [jnp.dot(r_alls[bb] * selmask, webT_ref[...],
                     preferred_element_type=f32) for bb in range(BB)]
    bias_ts = [jnp.dot(pool, bmats[bb], preferred_element_type=f32)
               for bb in range(BB)]
    bmaxs = [jnp.max(bt, axis=0, keepdims=True) for bt in bias_ts]
    expbs = [jnp.exp(bias_ts[bb] - bmaxs[bb]) for bb in range(BB)]

    bhs = [(bb, h) for bb in range(BB) for h in range(H)]
    gs = [None] * len(bhs)
    for e in range(E):
        for i, (bb, h) in enumerate(bhs):
            w = expbs[bb][e:e + 1, h:h + 1]
            me = mstacks[bb][e * L:(e + 1) * L, :]
            gs[i] = me * w if gs[i] is None else gs[i] + me * w
    shs = [jax.lax.dot_general(
        q[bb * L:(bb + 1) * L, h * Dh:(h + 1) * Dh],
        k[bb * L:(bb + 1) * L, h * Dh:(h + 1) * Dh],
        (((1,), (1,)), ((), ())), preferred_element_type=f32)
        for (bb, h) in bhs]
    ts = [shs[i] * scale + jnp.log(gs[i]) for i in range(len(bhs))]
    ms = [jnp.max(t, axis=1, keepdims=True) for t in ts]
    ps = [jnp.exp(ts[i] - ms[i]) for i in range(len(bhs))]
    sums = [jnp.sum(p, axis=1, keepdims=True) for p in ps]
    attns = [ps[i] / sums[i] for i in range(len(bhs))]
    ctxs = [jnp.dot(attns[i], v[bb * L:(bb + 1) * L, h * Dh:(h + 1) * Dh],
                    preferred_element_type=f32)
            for i, (bb, h) in enumerate(bhs)]
    ctx_parts = [jnp.concatenate(ctxs[bb * H:(bb + 1) * H], axis=1)
                 for bb in range(BB)]
    ctx = jnp.concatenate(ctx_parts, axis=0)
    out2 = x2 + jnp.dot(ctx, woT_ref[...],
                        preferred_element_type=f32) + bo_ref[...]
    out_ref[...] = out2.reshape(BB, L, D)


def kernel(single, pair, coords, contact_ss, ln_w, ln_b, Wq, Wk, Wv, W_eb,
           Wo, bo):
    B, L, D = single.shape
    DP = pair.shape[-1]
    BB = _BB
    coordsT = jnp.transpose(coords, (0, 2, 1))
    args = (single, pair, coords, coordsT, contact_ss,
            ln_w.reshape(1, D), ln_b.reshape(1, D),
            Wq.T, Wk.T, Wv.T, W_eb.T, Wo.T, bo.reshape(1, D))
    in_specs = [
        pl.BlockSpec((BB, L, D), lambda b: (b, 0, 0)),
        pl.BlockSpec((BB, 1, L, DP), lambda b: (b, 0, 0, 0)),
        pl.BlockSpec((BB, L, 3), lambda b: (b, 0, 0)),
        pl.BlockSpec((BB, 3, L), lambda b: (b, 0, 0)),
        pl.BlockSpec((BB, L, L), lambda b: (b, 0, 0)),
        pl.BlockSpec((1, D), lambda b: (0, 0)),
        pl.BlockSpec((1, D), lambda b: (0, 0)),
        pl.BlockSpec((D, D), lambda b: (0, 0)),
        pl.BlockSpec((D, D), lambda b: (0, 0)),
        pl.BlockSpec((D, D), lambda b: (0, 0)),
        pl.BlockSpec((DP, W_eb.shape[0]), lambda b: (0, 0)),
        pl.BlockSpec((D, D), lambda b: (0, 0)),
        pl.BlockSpec((1, D), lambda b: (0, 0)),
    ]
    return pl.pallas_call(
        _body,
        grid=(B // BB,),
        in_specs=in_specs,
        out_specs=pl.BlockSpec((BB, L, D), lambda b: (b, 0, 0)),
        out_shape=jax.ShapeDtypeStruct((B, L, D), jnp.float32),
    )(*args)


# speedup vs baseline: 468.1504x; 1.0293x over previous
import jax
import jax.numpy as jnp
from jax.experimental import pallas as pl

_K_KNN = 12
_K_SS = 4
_BB = 16


def _body(single_ref, pair0_ref, coords_ref, coordsT_ref, css_ref,
          ln_w_ref, ln_b_ref, wqT_ref, wkT_ref, wvT_ref, webT_ref,
          woT_ref, bo_ref, out_ref):
    f32 = jnp.float32
    BB, L, D = single_ref.shape
    H = webT_ref.shape[1]
    Dh = D // H
    E = _K_KNN + _K_SS
    R = BB * L
    scale = f32(Dh ** -0.5)

    x2 = single_ref[...].reshape(R, D)
    mu = jnp.mean(x2, axis=1, keepdims=True)
    xc = x2 - mu
    var = jnp.mean(xc * xc, axis=1, keepdims=True)
    xn = xc / jnp.sqrt(var + 1e-5) * ln_w_ref[...] + ln_b_ref[...]

    q = jnp.dot(xn, wqT_ref[...], preferred_element_type=f32)
    k = jnp.dot(xn, wkT_ref[...], preferred_element_type=f32)
    v = jnp.dot(xn, wvT_ref[...], preferred_element_type=f32)

    lanesR = jax.lax.broadcasted_iota(jnp.int32, (R, L), 1)
    rowsR = jax.lax.broadcasted_iota(jnp.int32, (R, L), 0)
    eyeR = jnp.where((rowsR % L) == lanesR, f32(1e9), f32(0.0))

    s2 = None
    for a in range(3):
        colv = coords_ref[...].reshape(R, 3)[:, a:a + 1]
        rowv = jnp.broadcast_to(coordsT_ref[:, a:a + 1, :],
                                (BB, L, L)).reshape(R, L)
        dc = colv - rowv
        s2 = dc * dc if s2 is None else s2 + dc * dc
    negd = -(jnp.sqrt(s2) + eyeR)
    css2 = css_ref[...].reshape(R, L)

    neg_inf = f32(-jnp.inf)

    def topk_cols(vals, kk):
        cols = []
        vc = vals
        for _ in range(kk):
            am = jnp.argmax(vc, axis=1, keepdims=True).astype(jnp.int32)
            cols.append(am)
            vc = jnp.where(lanesR == am, neg_inf, vc)
        return cols

    both = jnp.concatenate([negd, css2], axis=0)
    lanes2R = jax.lax.broadcasted_iota(jnp.int32, (2 * R, L), 1)
    vc2 = both
    cols2 = []
    for _ in range(_K_SS):
        am = jnp.argmax(vc2, axis=1, keepdims=True).astype(jnp.int32)
        cols2.append(am)
        vc2 = jnp.where(lanes2R == am, neg_inf, vc2)
    knn_cols = [c[:R] for c in cols2] + topk_cols(vc2[:R], _K_KNN - _K_SS)
    ss_cols = [c[R:] for c in cols2]
    cols = knn_cols + ss_cols

    lanesEL = jax.lax.broadcasted_iota(jnp.int32, (E * L, L), 1)
    rowsEL = jax.lax.broadcasted_iota(jnp.int32, (E * L, L), 0)
    selmask = (lanesEL == (16 * ((rowsEL % L) % 4) + rowsEL // L)).astype(f32)
    r2 = jax.lax.broadcasted_iota(jnp.int32, (E, E * L), 1)
    e2i = jax.lax.broadcasted_iota(jnp.int32, (E, E * L), 0)
    pool = (((r2 % L) // 4) == e2i).astype(f32)

    catcols = [jnp.concatenate([cols[e][bb * L:(bb + 1) * L] for e in range(E)],
                               axis=0) for bb in range(BB)]
    mstacks = [(cc == lanesEL).astype(f32) for cc in catcols]
    r_alls = [jnp.dot(mstacks[bb], pair0_ref[bb, 0],
                      preferred_element_type=f32) for bb in range(BB)]
    bmats = [jnp.dot(r_alls[bb] * selmask, webT_ref[...],
                     preferred_element_type=f32) for bb in range(BB)]
    bias_ts = [jnp.dot(pool, bmats[bb], preferred_element_type=f32)
               for bb in range(BB)]
    bmaxs = [jnp.max(bt, axis=0, keepdims=True) for bt in bias_ts]
    expbs = [jnp.exp(bias_ts[bb] - bmaxs[bb]) for bb in range(BB)]

    bhs = [(bb, h) for bb in range(BB) for h in range(H)]
    gs = [None] * len(bhs)
    for e in range(E):
        for i, (bb, h) in enumerate(bhs):
            w = expbs[bb][e:e + 1, h:h + 1]
            me = mstacks[bb][e * L:(e + 1) * L, :]
            gs[i] = me * w if gs[i] is None else gs[i] + me * w
    shs = [jax.lax.dot_general(
        q[bb * L:(bb + 1) * L, h * Dh:(h + 1) * Dh],
        k[bb * L:(bb + 1) * L, h * Dh:(h + 1) * Dh],
        (((1,), (1,)), ((), ())), preferred_element_type=f32)
        for (bb, h) in bhs]
    ts = [shs[i] * scale + jnp.log(gs[i]) for i in range(len(bhs))]
    ms = [jnp.max(t, axis=1, keepdims=True) for t in ts]
    ps = [jnp.exp(ts[i] - ms[i]) for i in range(len(bhs))]
    sums = [jnp.sum(p, axis=1, keepdims=True) for p in ps]
    attns = [ps[i] / sums[i] for i in range(len(bhs))]
    ctxs = [jnp.dot(attns[i], v[bb * L:(bb + 1) * L, h * Dh:(h + 1) * Dh],
                    preferred_element_type=f32)
            for i, (bb, h) in enumerate(bhs)]
    ctx_parts = [jnp.concatenate(ctxs[bb * H:(bb + 1) * H], axis=1)
                 for bb in range(BB)]
    ctx = jnp.concatenate(ctx_parts, axis=0)
    out2 = x2 + jnp.dot(ctx, woT_ref[...],
                        preferred_element_type=f32) + bo_ref[...]
    out_ref[...] = out2.reshape(BB, L, D)


def kernel(single, pair, coords, contact_ss, ln_w, ln_b, Wq, Wk, Wv, W_eb,
           Wo, bo):
    B, L, D = single.shape
    DP = pair.shape[-1]
    BB = _BB
    coordsT = jnp.transpose(coords, (0, 2, 1))
    args = (single, pair, coords, coordsT, contact_ss,
            ln_w.reshape(1, D), ln_b.reshape(1, D),
            Wq.T, Wk.T, Wv.T, W_eb.T, Wo.T, bo.reshape(1, D))
    in_specs = [
        pl.BlockSpec((BB, L, D), lambda b: (b, 0, 0)),
        pl.BlockSpec((BB, 1, L, DP), lambda b: (b, 0, 0, 0)),
        pl.BlockSpec((BB, L, 3), lambda b: (b, 0, 0)),
        pl.BlockSpec((BB, 3, L), lambda b: (b, 0, 0)),
        pl.BlockSpec((BB, L, L), lambda b: (b, 0, 0)),
        pl.BlockSpec((1, D), lambda b: (0, 0)),
        pl.BlockSpec((1, D), lambda b: (0, 0)),
        pl.BlockSpec((D, D), lambda b: (0, 0)),
        pl.BlockSpec((D, D), lambda b: (0, 0)),
        pl.BlockSpec((D, D), lambda b: (0, 0)),
        pl.BlockSpec((DP, W_eb.shape[0]), lambda b: (0, 0)),
        pl.BlockSpec((D, D), lambda b: (0, 0)),
        pl.BlockSpec((1, D), lambda b: (0, 0)),
    ]
    return pl.pallas_call(
        _body,
        grid=(B // BB,),
        in_specs=in_specs,
        out_specs=pl.BlockSpec((BB, L, D), lambda b: (b, 0, 0)),
        out_shape=jax.ShapeDtypeStruct((B, L, D), jnp.float32),
    )(*args)


# speedup vs baseline: 477.6719x; 1.0203x over previous
import jax
import jax.numpy as jnp
import numpy as np
from jax.experimental import pallas as pl

_K_KNN = 12
_K_SS = 4
_BB = 16


def _body(single_ref, pair0_ref, coords_ref, coordsT_ref, css_ref,
          ln_w_ref, ln_b_ref, wqT_ref, wkT_ref, wvT_ref, webT_ref,
          woT_ref, bo_ref, eye_ref, selmask_ref, pool_ref, out_ref):
    f32 = jnp.float32
    BB, L, D = single_ref.shape
    H = webT_ref.shape[1]
    Dh = D // H
    E = _K_KNN + _K_SS
    R = BB * L
    scale = f32(Dh ** -0.5)

    x2 = single_ref[...].reshape(R, D)
    mu = jnp.mean(x2, axis=1, keepdims=True)
    xc = x2 - mu
    var = jnp.mean(xc * xc, axis=1, keepdims=True)
    xn = xc / jnp.sqrt(var + 1e-5) * ln_w_ref[...] + ln_b_ref[...]

    q = jnp.dot(xn, wqT_ref[...], preferred_element_type=f32)
    k = jnp.dot(xn, wkT_ref[...], preferred_element_type=f32)
    v = jnp.dot(xn, wvT_ref[...], preferred_element_type=f32)

    lanesR = jax.lax.broadcasted_iota(jnp.int32, (R, L), 1)

    s2 = None
    for a in range(3):
        colv = coords_ref[...].reshape(R, 3)[:, a:a + 1]
        rowv = jnp.broadcast_to(coordsT_ref[:, a:a + 1, :],
                                (BB, L, L)).reshape(R, L)
        dc = colv - rowv
        s2 = dc * dc if s2 is None else s2 + dc * dc
    negd = -(jnp.sqrt(s2) + eye_ref[...])
    css2 = css_ref[...].reshape(R, L)

    neg_inf = f32(-jnp.inf)

    def topk_cols(vals, kk):
        cols = []
        vc = vals
        for _ in range(kk):
            am = jnp.argmax(vc, axis=1, keepdims=True).astype(jnp.int32)
            cols.append(am)
            vc = jnp.where(lanesR == am, neg_inf, vc)
        return cols

    both = jnp.concatenate([negd, css2], axis=0)
    lanes2R = jax.lax.broadcasted_iota(jnp.int32, (2 * R, L), 1)
    vc2 = both
    cols2 = []
    for _ in range(_K_SS):
        am = jnp.argmax(vc2, axis=1, keepdims=True).astype(jnp.int32)
        cols2.append(am)
        vc2 = jnp.where(lanes2R == am, neg_inf, vc2)
    knn_cols = [c[:R] for c in cols2] + topk_cols(vc2[:R], _K_KNN - _K_SS)
    ss_cols = [c[R:] for c in cols2]
    cols = knn_cols + ss_cols

    lanesEL = jax.lax.broadcasted_iota(jnp.int32, (E * L, L), 1)

    catcols = [jnp.concatenate([cols[e][bb * L:(bb + 1) * L] for e in range(E)],
                               axis=0) for bb in range(BB)]
    mstacks = [(cc == lanesEL).astype(f32) for cc in catcols]
    r_alls = [jnp.dot(mstacks[bb], pair0_ref[bb, 0],
                      preferred_element_type=f32) for bb in range(BB)]
    bmats = [jnp.dot(r_alls[bb] * selmask_ref[...], webT_ref[...],
                     preferred_element_type=f32) for bb in range(BB)]
    bias_ts = [jnp.dot(pool_ref[...], bmats[bb], preferred_element_type=f32)
               for bb in range(BB)]
    bmaxs = [jnp.max(bt, axis=0, keepdims=True) for bt in bias_ts]
    expbs = [jnp.exp(bias_ts[bb] - bmaxs[bb]) for bb in range(BB)]

    bhs = [(bb, h) for bb in range(BB) for h in range(H)]
    shs = [jax.lax.dot_general(
        q[bb * L:(bb + 1) * L, h * Dh:(h + 1) * Dh],
        k[bb * L:(bb + 1) * L, h * Dh:(h + 1) * Dh],
        (((1,), (1,)), ((), ())), preferred_element_type=f32)
        for (bb, h) in bhs]
    sscal = [sh * scale for sh in shs]
    ms = [jnp.max(t, axis=1, keepdims=True) for t in sscal]
    es = [jnp.exp(sscal[i] - ms[i]) for i in range(len(bhs))]
    gs = [None] * len(bhs)
    for bb in range(BB):
        for e in range(E):
            me = mstacks[bb][e * L:(e + 1) * L, :]
            for h in range(H):
                i = bb * H + h
                w = expbs[bb][e:e + 1, h:h + 1]
                gs[i] = me * w if gs[i] is None else gs[i] + me * w
    ps = [es[i] * gs[i] for i in range(len(bhs))]
    sums = [jnp.sum(p, axis=1, keepdims=True) for p in ps]
    attns = [ps[i] / sums[i] for i in range(len(bhs))]
    ctxs = [jnp.dot(attns[i], v[bb * L:(bb + 1) * L, h * Dh:(h + 1) * Dh],
                    preferred_element_type=f32)
            for i, (bb, h) in enumerate(bhs)]
    ctx_parts = [jnp.concatenate(ctxs[bb * H:(bb + 1) * H], axis=1)
                 for bb in range(BB)]
    ctx = jnp.concatenate(ctx_parts, axis=0)
    out2 = x2 + jnp.dot(ctx, woT_ref[...],
                        preferred_element_type=f32) + bo_ref[...]
    out_ref[...] = out2.reshape(BB, L, D)


def kernel(single, pair, coords, contact_ss, ln_w, ln_b, Wq, Wk, Wv, W_eb,
           Wo, bo):
    B, L, D = single.shape
    DP = pair.shape[-1]
    BB = _BB
    E = _K_KNN + _K_SS
    R = BB * L
    coordsT = jnp.transpose(coords, (0, 2, 1))
    ii = np.arange(R)[:, None] % L
    jj = np.arange(L)[None, :]
    eye_big = jnp.asarray(np.where(ii == jj, 1e9, 0.0).astype(np.float32))
    rEL = np.arange(E * L)[:, None]
    selmask = jnp.asarray(
        (jj == (16 * ((rEL % L) % 4) + rEL // L)).astype(np.float32))
    rp = np.arange(E * L)[None, :]
    ep = np.arange(E)[:, None]
    pool = jnp.asarray((((rp % L) // 4) == ep).astype(np.float32))
    args = (single, pair, coords, coordsT, contact_ss,
            ln_w.reshape(1, D), ln_b.reshape(1, D),
            Wq.T, Wk.T, Wv.T, W_eb.T, Wo.T, bo.reshape(1, D),
            eye_big, selmask, pool)
    in_specs = [
        pl.BlockSpec((BB, L, D), lambda b: (b, 0, 0)),
        pl.BlockSpec((BB, 1, L, DP), lambda b: (b, 0, 0, 0)),
        pl.BlockSpec((BB, L, 3), lambda b: (b, 0, 0)),
        pl.BlockSpec((BB, 3, L), lambda b: (b, 0, 0)),
        pl.BlockSpec((BB, L, L), lambda b: (b, 0, 0)),
        pl.BlockSpec((1, D), lambda b: (0, 0)),
        pl.BlockSpec((1, D), lambda b: (0, 0)),
        pl.BlockSpec((D, D), lambda b: (0, 0)),
        pl.BlockSpec((D, D), lambda b: (0, 0)),
        pl.BlockSpec((D, D), lambda b: (0, 0)),
        pl.BlockSpec((DP, W_eb.shape[0]), lambda b: (0, 0)),
        pl.BlockSpec((D, D), lambda b: (0, 0)),
        pl.BlockSpec((1, D), lambda b: (0, 0)),
        pl.BlockSpec((R, L), lambda b: (0, 0)),
        pl.BlockSpec((E * L, L), lambda b: (0, 0)),
        pl.BlockSpec((E, E * L), lambda b: (0, 0)),
    ]
    return pl.pallas_call(
        _body,
        grid=(B // BB,),
        in_specs=in_specs,
        out_specs=pl.BlockSpec((BB, L, D), lambda b: (b, 0, 0)),
        out_shape=jax.ShapeDtypeStruct((B, L, D), jnp.float32),
    )(*args)
